# baseline (speedup 1.0000x reference)
import functools

import jax
import jax.numpy as jnp
from jax import lax
from jax.experimental import pallas as pl
from jax.experimental.pallas import tpu as pltpu
from jax.experimental.pallas import tpu_sc as plsc

N_FIELDS = 26
VOCAB_P1 = 100001
EMBED = 32
BATCH = 16384

NVB = 800
VP = NVB * 128

NC = 2
NS = 16
NW = NC * NS
VB_PW = NVB // NW
V_PW = VB_PW * 128
BPW = BATCH // NW
CHUNK = 128
NCHUNK = BPW // CHUNK

SUBS = ((0, 13), (13, 12))


def _cp_body(tt_ref, o_ref):
    o_ref[...] = tt_ref[0]


def _tp_body(ztab_hbm, zrow_hbm, tab_v, out_t):
    wid = lax.axis_index("s") * NC + lax.axis_index("c")
    iota = lax.iota(jnp.int32, 16)

    def per_field(f, _):
        for sub_off, sub_nvb in SUBS:
            nrows_in = sub_nvb * EMBED
            nv = sub_nvb * 128
            in_base = (f * NVB + wid * VB_PW + sub_off) * EMBED
            pltpu.sync_copy(
                ztab_hbm.at[pl.ds(in_base, nrows_in)],
                tab_v.at[pl.ds(0, nrows_in)],
            )

            def per_group(p, _):
                v16 = p * 16 + iota
                row = (v16 >> 7) << 5
                col = v16 & 127
                for r in range(EMBED):
                    vals = plsc.load_gather(tab_v, [row + r, col])
                    plsc.store_scatter(out_t, [v16 * EMBED + r], vals)
                return 0

            lax.fori_loop(0, nv // 16, per_group, 0)
            out_base = (f * VP + wid * V_PW + sub_off * 128) * EMBED
            pltpu.sync_copy(
                out_t.at[pl.ds(0, nv * EMBED)],
                zrow_hbm.at[pl.ds(out_base, nv * EMBED)],
            )
        return 0

    lax.fori_loop(0, N_FIELDS, per_field, 0)


def _sc_body(x3_hbm, tflat_hbm, *rest):
    outs = rest[:N_FIELDS]
    idx_v, rows_v, gsem = rest[N_FIELDS:]
    wid = lax.axis_index("s") * NC + lax.axis_index("c")
    base = wid * BPW
    for i in range(N_FIELDS):
        pltpu.sync_copy(x3_hbm.at[i, pl.ds(wid * NCHUNK, NCHUNK)], idx_v)
        descs = []
        for c in range(NCHUNK):
            descs.append(
                pltpu.async_copy(
                    tflat_hbm.at[pl.ds(i * VP, VP)].at[idx_v.at[c]],
                    rows_v.at[pl.ds(c * CHUNK, CHUNK)],
                    gsem,
                )
            )
        for d in descs:
            d.wait()
        pltpu.sync_copy(rows_v, outs[i].at[pl.ds(base, BPW)])


@jax.jit
def _embed_all(x3, tablesT):
    ztab = pl.pallas_call(
        _cp_body,
        grid=(N_FIELDS, NVB),
        in_specs=[pl.BlockSpec((1, EMBED, 128), lambda f, vb: (f, 0, vb))],
        out_specs=pl.BlockSpec((EMBED, 128), lambda f, vb: (f * NVB + vb, 0)),
        out_shape=jax.ShapeDtypeStruct((N_FIELDS * NVB * EMBED, 128), jnp.float32),
    )(tablesT)

    mesh = plsc.VectorSubcoreMesh(core_axis_name="c", subcore_axis_name="s")
    zrow = pl.kernel(
        _tp_body,
        mesh=mesh,
        out_type=jax.ShapeDtypeStruct((N_FIELDS * VP * EMBED,), jnp.float32),
        scratch_types=[
            pltpu.VMEM((SUBS[0][1] * EMBED, 128), jnp.float32),
            pltpu.VMEM((SUBS[0][1] * 128 * EMBED,), jnp.float32),
        ],
        compiler_params=pltpu.CompilerParams(
            use_tc_tiling_on_sc=False, needs_layout_passes=False
        ),
    )(ztab)
    z2d = zrow.reshape(N_FIELDS * VP, EMBED)

    f = pl.kernel(
        _sc_body,
        mesh=mesh,
        out_type=[jax.ShapeDtypeStruct((BATCH, EMBED), jnp.float32)] * N_FIELDS,
        scratch_types=[
            pltpu.VMEM((NCHUNK, CHUNK), jnp.int32),
            pltpu.VMEM((BPW, EMBED), jnp.float32),
            pltpu.SemaphoreType.DMA,
        ],
        compiler_params=pltpu.CompilerParams(use_tc_tiling_on_sc=False),
    )
    return f(x3, z2d)


def kernel(x, tables):
    x3 = x.T.reshape(N_FIELDS, BATCH // CHUNK, CHUNK)
    tablesT = jnp.transpose(tables, (0, 2, 1))
    return tuple(_embed_all(x3, tablesT))


# speedup vs baseline: 6.5159x; 6.5159x over previous
import functools

import jax
import jax.numpy as jnp
from jax import lax
from jax.experimental import pallas as pl
from jax.experimental.pallas import tpu as pltpu
from jax.experimental.pallas import tpu_sc as plsc

N_FIELDS = 26
VOCAB_P1 = 100001
EMBED = 32
BATCH = 16384

NVB = 800
VP = NVB * 128

NC = 2
NS = 16
NW = NC * NS
VB_PW = NVB // NW
V_PW = VB_PW * 128
BPW = BATCH // NW
CHUNK = 128
NCHUNK = BPW // CHUNK

SUBS = ((0, 13), (13, 12))


def _cp_body(tt_ref, o_ref):
    t = tt_ref[0]
    o_ref[...] = jnp.concatenate(
        [t[:, k * 128:(k + 1) * 128] for k in range(32)], axis=0
    )


def _tp_body(ztab_hbm, zrow_hbm, tab_v, out_t):
    wid = lax.axis_index("s") * NC + lax.axis_index("c")
    iota = lax.iota(jnp.int32, 16)

    def per_field(f, _):
        for sub_off, sub_nvb in SUBS:
            nrows_in = sub_nvb * EMBED
            nv = sub_nvb * 128
            in_base = (f * NVB + wid * VB_PW + sub_off) * EMBED
            pltpu.sync_copy(
                ztab_hbm.at[pl.ds(in_base, nrows_in)],
                tab_v.at[pl.ds(0, nrows_in)],
            )

            def per_group(p, _):
                v16 = p * 16 + iota
                row = (v16 >> 7) << 5
                col = v16 & 127
                for j in range(EMBED):
                    r_vec = (iota + j) & (EMBED - 1)
                    vals = plsc.load_gather(tab_v, [row + r_vec, col])
                    plsc.store_scatter(out_t, [v16 * EMBED + r_vec], vals)
                return 0

            lax.fori_loop(0, nv // 16, per_group, 0)
            out_base = (f * VP + wid * V_PW + sub_off * 128) * EMBED
            pltpu.sync_copy(
                out_t.at[pl.ds(0, nv * EMBED)],
                zrow_hbm.at[pl.ds(out_base, nv * EMBED)],
            )
        return 0

    lax.fori_loop(0, N_FIELDS, per_field, 0)


def _sc_body(x3_hbm, tflat_hbm, *rest):
    outs = rest[:N_FIELDS]
    idx_v, rows_v, gsem = rest[N_FIELDS:]
    wid = lax.axis_index("s") * NC + lax.axis_index("c")
    base = wid * BPW
    for i in range(N_FIELDS):
        pltpu.sync_copy(x3_hbm.at[i, pl.ds(wid * NCHUNK, NCHUNK)], idx_v)
        descs = []
        for c in range(NCHUNK):
            descs.append(
                pltpu.async_copy(
                    tflat_hbm.at[pl.ds(i * VP, VP)].at[idx_v.at[c]],
                    rows_v.at[pl.ds(c * CHUNK, CHUNK)],
                    gsem,
                )
            )
        for d in descs:
            d.wait()
        pltpu.sync_copy(rows_v, outs[i].at[pl.ds(base, BPW)])


@jax.jit
def _embed_all(x3, tablesT):
    ztab = pl.pallas_call(
        _cp_body,
        grid=(N_FIELDS, NVB // 32),
        in_specs=[pl.BlockSpec((1, EMBED, 4096), lambda f, j: (f, 0, j))],
        out_specs=pl.BlockSpec((32 * EMBED, 128), lambda f, j: (f * (NVB // 32) + j, 0)),
        out_shape=jax.ShapeDtypeStruct((N_FIELDS * NVB * EMBED, 128), jnp.float32),
    )(tablesT)

    mesh = plsc.VectorSubcoreMesh(core_axis_name="c", subcore_axis_name="s")
    zrow = pl.kernel(
        _tp_body,
        mesh=mesh,
        out_type=jax.ShapeDtypeStruct((N_FIELDS * VP * EMBED,), jnp.float32),
        scratch_types=[
            pltpu.VMEM((SUBS[0][1] * EMBED, 128), jnp.float32),
            pltpu.VMEM((SUBS[0][1] * 128 * EMBED,), jnp.float32),
        ],
        compiler_params=pltpu.CompilerParams(
            use_tc_tiling_on_sc=False, needs_layout_passes=False
        ),
    )(ztab)
    z2d = zrow.reshape(N_FIELDS * VP, EMBED)

    f = pl.kernel(
        _sc_body,
        mesh=mesh,
        out_type=[jax.ShapeDtypeStruct((BATCH, EMBED), jnp.float32)] * N_FIELDS,
        scratch_types=[
            pltpu.VMEM((NCHUNK, CHUNK), jnp.int32),
            pltpu.VMEM((BPW, EMBED), jnp.float32),
            pltpu.SemaphoreType.DMA,
        ],
        compiler_params=pltpu.CompilerParams(use_tc_tiling_on_sc=False),
    )
    return f(x3, z2d)


def kernel(x, tables):
    x3 = x.T.reshape(N_FIELDS, BATCH // CHUNK, CHUNK)
    tablesT = jnp.transpose(tables, (0, 2, 1))
    return tuple(_embed_all(x3, tablesT))


# speedup vs baseline: 8.0061x; 1.2287x over previous
import functools

import jax
import jax.numpy as jnp
from jax import lax
from jax.experimental import pallas as pl
from jax.experimental.pallas import tpu as pltpu
from jax.experimental.pallas import tpu_sc as plsc

N_FIELDS = 26
VOCAB_P1 = 100001
EMBED = 32
BATCH = 16384

NVB = 800
VP = NVB * 128

NC = 2
NS = 16
NW = NC * NS
VB_PW = NVB // NW
V_PW = VB_PW * 128
BPW = BATCH // NW
CHUNK = 128
NCHUNK = BPW // CHUNK

SUBS = ((0, 13), (13, 12))

GROUP_SIZES = (7, 7, 6, 6)


def _cp_body(tt_ref, o_ref):
    t = tt_ref[0]
    o_ref[...] = jnp.concatenate(
        [t[:, k * 128:(k + 1) * 128] for k in range(32)], axis=0
    )


def _make_tp_body(nf):
    def _tp_body(ztab_hbm, zrow_hbm, tab_v, out_t):
        wid = lax.axis_index("s") * NC + lax.axis_index("c")
        iota = lax.iota(jnp.int32, 16)

        def per_field(f, _):
            for sub_off, sub_nvb in SUBS:
                nrows_in = sub_nvb * EMBED
                nv = sub_nvb * 128
                in_base = (f * NVB + wid * VB_PW + sub_off) * EMBED
                pltpu.sync_copy(
                    ztab_hbm.at[pl.ds(in_base, nrows_in)],
                    tab_v.at[pl.ds(0, nrows_in)],
                )

                def per_group(p, _):
                    v16 = p * 16 + iota
                    row = (v16 >> 7) << 5
                    col = v16 & 127
                    for j in range(EMBED):
                        r_vec = (iota + j) & (EMBED - 1)
                        vals = plsc.load_gather(tab_v, [row + r_vec, col])
                        plsc.store_scatter(out_t, [v16 * EMBED + r_vec], vals)
                    return 0

                lax.fori_loop(0, nv // 16, per_group, 0)
                out_base = (f * VP + wid * V_PW + sub_off * 128) * EMBED
                pltpu.sync_copy(
                    out_t.at[pl.ds(0, nv * EMBED)],
                    zrow_hbm.at[pl.ds(out_base, nv * EMBED)],
                )
            return 0

        lax.fori_loop(0, nf, per_field, 0)

    return _tp_body


def _make_sc_body(nf, f0):
    def _sc_body(x3_hbm, tflat_hbm, *rest):
        outs = rest[:nf]
        idx_v, rows_v, gsem = rest[nf:]
        wid = lax.axis_index("s") * NC + lax.axis_index("c")
        base = wid * BPW
        for i in range(nf):
            pltpu.sync_copy(
                x3_hbm.at[f0 + i, pl.ds(wid * NCHUNK, NCHUNK)], idx_v
            )
            descs = []
            for c in range(NCHUNK):
                descs.append(
                    pltpu.async_copy(
                        tflat_hbm.at[pl.ds(i * VP, VP)].at[idx_v.at[c]],
                        rows_v.at[pl.ds(c * CHUNK, CHUNK)],
                        gsem,
                    )
                )
            for d in descs:
                d.wait()
            pltpu.sync_copy(rows_v, outs[i].at[pl.ds(base, BPW)])

    return _sc_body


@jax.jit
def _embed_all(x3, tablesT):
    mesh = plsc.VectorSubcoreMesh(core_axis_name="c", subcore_axis_name="s")
    outs = []
    f0 = 0
    for nf in GROUP_SIZES:
        ztab = pl.pallas_call(
            _cp_body,
            grid=(nf, NVB // 32),
            in_specs=[
                pl.BlockSpec(
                    (1, EMBED, 4096),
                    functools.partial(lambda f0, f, j: (f0 + f, 0, j), f0),
                )
            ],
            out_specs=pl.BlockSpec(
                (32 * EMBED, 128), lambda f, j: (f * (NVB // 32) + j, 0)
            ),
            out_shape=jax.ShapeDtypeStruct((nf * NVB * EMBED, 128), jnp.float32),
        )(tablesT)

        zrow = pl.kernel(
            _make_tp_body(nf),
            mesh=mesh,
            out_type=jax.ShapeDtypeStruct((nf * VP * EMBED,), jnp.float32),
            scratch_types=[
                pltpu.VMEM((SUBS[0][1] * EMBED, 128), jnp.float32),
                pltpu.VMEM((SUBS[0][1] * 128 * EMBED,), jnp.float32),
            ],
            compiler_params=pltpu.CompilerParams(
                use_tc_tiling_on_sc=False, needs_layout_passes=False
            ),
        )(ztab)
        z2d = zrow.reshape(nf * VP, EMBED)

        g = pl.kernel(
            _make_sc_body(nf, f0),
            mesh=mesh,
            out_type=[jax.ShapeDtypeStruct((BATCH, EMBED), jnp.float32)] * nf,
            scratch_types=[
                pltpu.VMEM((NCHUNK, CHUNK), jnp.int32),
                pltpu.VMEM((BPW, EMBED), jnp.float32),
                pltpu.SemaphoreType.DMA,
            ],
            compiler_params=pltpu.CompilerParams(use_tc_tiling_on_sc=False),
        )
        outs.extend(g(x3, z2d))
        f0 += nf
    return outs


def kernel(x, tables):
    x3 = x.T.reshape(N_FIELDS, BATCH // CHUNK, CHUNK)
    tablesT = jnp.transpose(tables, (0, 2, 1))
    return tuple(_embed_all(x3, tablesT))


# speedup vs baseline: 8.1086x; 1.0128x over previous
import functools

import jax
import jax.numpy as jnp
from jax import lax
from jax.experimental import pallas as pl
from jax.experimental.pallas import tpu as pltpu
from jax.experimental.pallas import tpu_sc as plsc

N_FIELDS = 26
VOCAB_P1 = 100001
EMBED = 32
BATCH = 16384

NVB = 800
VP = NVB * 128

NC = 2
NS = 16
NW = NC * NS
VB_PW = NVB // NW
V_PW = VB_PW * 128
BPW = BATCH // NW
CHUNK = 128
NCHUNK = BPW // CHUNK

SUBS = ((0, 13), (13, 12))

GROUP_SIZES = (5, 5, 4, 4, 4, 4)


def _cp_body(tt_ref, o_ref):
    t = tt_ref[0]
    o_ref[...] = jnp.concatenate(
        [t[:, k * 128:(k + 1) * 128] for k in range(32)], axis=0
    )


def _make_tp_body(nf):
    def _tp_body(ztab_hbm, zrow_hbm, tab_v, out_t):
        wid = lax.axis_index("s") * NC + lax.axis_index("c")
        iota = lax.iota(jnp.int32, 16)

        def per_field(f, _):
            for sub_off, sub_nvb in SUBS:
                nrows_in = sub_nvb * EMBED
                nv = sub_nvb * 128
                in_base = (f * NVB + wid * VB_PW + sub_off) * EMBED
                pltpu.sync_copy(
                    ztab_hbm.at[pl.ds(in_base, nrows_in)],
                    tab_v.at[pl.ds(0, nrows_in)],
                )

                def per_group(p, _):
                    v16 = p * 16 + iota
                    row = (v16 >> 7) << 5
                    col = v16 & 127
                    for j in range(EMBED):
                        r_vec = (iota + j) & (EMBED - 1)
                        vals = plsc.load_gather(tab_v, [row + r_vec, col])
                        plsc.store_scatter(out_t, [v16 * EMBED + r_vec], vals)
                    return 0

                lax.fori_loop(0, nv // 16, per_group, 0)
                out_base = (f * VP + wid * V_PW + sub_off * 128) * EMBED
                pltpu.sync_copy(
                    out_t.at[pl.ds(0, nv * EMBED)],
                    zrow_hbm.at[pl.ds(out_base, nv * EMBED)],
                )
            return 0

        lax.fori_loop(0, nf, per_field, 0)

    return _tp_body


def _make_sc_body(nf, f0):
    def _sc_body(x3_hbm, tflat_hbm, *rest):
        outs = rest[:nf]
        idx_v, rows_v, gsem = rest[nf:]
        wid = lax.axis_index("s") * NC + lax.axis_index("c")
        base = wid * BPW
        for i in range(nf):
            pltpu.sync_copy(
                x3_hbm.at[f0 + i, pl.ds(wid * NCHUNK, NCHUNK)], idx_v
            )
            descs = []
            for c in range(NCHUNK):
                descs.append(
                    pltpu.async_copy(
                        tflat_hbm.at[pl.ds(i * VP, VP)].at[idx_v.at[c]],
                        rows_v.at[pl.ds(c * CHUNK, CHUNK)],
                        gsem,
                    )
                )
            for d in descs:
                d.wait()
            pltpu.sync_copy(rows_v, outs[i].at[pl.ds(base, BPW)])

    return _sc_body


@jax.jit
def _embed_all(x3, tablesT):
    mesh = plsc.VectorSubcoreMesh(core_axis_name="c", subcore_axis_name="s")
    outs = []
    f0 = 0
    for nf in GROUP_SIZES:
        ztab = pl.pallas_call(
            _cp_body,
            grid=(nf, NVB // 32),
            in_specs=[
                pl.BlockSpec(
                    (1, EMBED, 4096),
                    functools.partial(lambda f0, f, j: (f0 + f, 0, j), f0),
                )
            ],
            out_specs=pl.BlockSpec(
                (32 * EMBED, 128), lambda f, j: (f * (NVB // 32) + j, 0)
            ),
            out_shape=jax.ShapeDtypeStruct((nf * NVB * EMBED, 128), jnp.float32),
        )(tablesT)

        zrow = pl.kernel(
            _make_tp_body(nf),
            mesh=mesh,
            out_type=jax.ShapeDtypeStruct((nf * VP * EMBED,), jnp.float32),
            scratch_types=[
                pltpu.VMEM((SUBS[0][1] * EMBED, 128), jnp.float32),
                pltpu.VMEM((SUBS[0][1] * 128 * EMBED,), jnp.float32),
            ],
            compiler_params=pltpu.CompilerParams(
                use_tc_tiling_on_sc=False, needs_layout_passes=False
            ),
        )(ztab)
        z2d = zrow.reshape(nf * VP, EMBED)

        g = pl.kernel(
            _make_sc_body(nf, f0),
            mesh=mesh,
            out_type=[jax.ShapeDtypeStruct((BATCH, EMBED), jnp.float32)] * nf,
            scratch_types=[
                pltpu.VMEM((NCHUNK, CHUNK), jnp.int32),
                pltpu.VMEM((BPW, EMBED), jnp.float32),
                pltpu.SemaphoreType.DMA,
            ],
            compiler_params=pltpu.CompilerParams(use_tc_tiling_on_sc=False),
        )
        outs.extend(g(x3, z2d))
        f0 += nf
    return outs


def kernel(x, tables):
    x3 = x.T.reshape(N_FIELDS, BATCH // CHUNK, CHUNK)
    tablesT = jnp.transpose(tables, (0, 2, 1))
    return tuple(_embed_all(x3, tablesT))


# speedup vs baseline: 9.6828x; 1.1941x over previous
import functools

import jax
import jax.numpy as jnp
from jax import lax
from jax.experimental import pallas as pl
from jax.experimental.pallas import tpu as pltpu
from jax.experimental.pallas import tpu_sc as plsc

N_FIELDS = 26
VOCAB_P1 = 100001
EMBED = 32
BATCH = 16384

NVB = 800
VP = NVB * 128

NC = 2
NS = 16
NW = NC * NS
VB_PW = NVB // NW
V_PW = VB_PW * 128
BPW = BATCH // NW
CHUNK = 128
NCHUNK = BPW // CHUNK

PH_OFF = (0, 7, 13, 19)
PH_NVB = (7, 6, 6, 6)
PH_MAX = 7

GROUP_SIZES = (5, 5, 4, 4, 4, 4)


def _cp_body(tt_ref, o_ref):
    t = tt_ref[0]
    o_ref[...] = jnp.concatenate(
        [t[:, k * 128:(k + 1) * 128] for k in range(32)], axis=0
    )


def _make_tp_body(nf):
    def _tp_body(ztab_hbm, zrow_hbm, tab0, tab1, out0, out1, lsem, ssem):
        wid = lax.axis_index("s") * NC + lax.axis_index("c")
        iota = lax.iota(jnp.int32, 16)
        tabs = (tab0, tab1)
        outs_t = (out0, out1)

        def load_start(f, ph):
            nrows = PH_NVB[ph] * EMBED
            in_base = (f * NVB + wid * VB_PW + PH_OFF[ph]) * EMBED
            pltpu.async_copy(
                ztab_hbm.at[pl.ds(in_base, nrows)],
                tabs[ph & 1].at[pl.ds(0, nrows)],
                lsem,
            )

        def load_drain(ph):
            nrows = PH_NVB[ph] * EMBED
            pltpu.make_async_copy(
                ztab_hbm.at[pl.ds(0, nrows)],
                tabs[ph & 1].at[pl.ds(0, nrows)],
                lsem,
            ).wait()

        def store_drain(ph):
            nwords = PH_NVB[ph] * 128 * EMBED
            pltpu.make_async_copy(
                zrow_hbm.at[pl.ds(0, nwords)],
                outs_t[ph & 1].at[pl.ds(0, nwords)],
                ssem,
            ).wait()

        load_start(0, 0)

        def per_field(f, _):
            for ph in range(4):
                tab_v = tabs[ph & 1]
                out_t = outs_t[ph & 1]
                nv = PH_NVB[ph] * 128
                if ph < 3:
                    load_start(f, ph + 1)
                else:
                    @pl.when(f < nf - 1)
                    def _():
                        load_start(f + 1, 0)
                load_drain(ph)
                if ph >= 2:
                    store_drain(ph - 2)
                else:
                    @pl.when(f > 0)
                    def _():
                        store_drain(ph + 2)

                def per_group(p, _):
                    v16 = p * 16 + iota
                    row = (v16 >> 7) << 5
                    col = v16 & 127
                    for j in range(EMBED):
                        r_vec = (iota + j) & (EMBED - 1)
                        vals = plsc.load_gather(tab_v, [row + r_vec, col])
                        plsc.store_scatter(out_t, [v16 * EMBED + r_vec], vals)
                    return 0

                lax.fori_loop(0, nv // 16, per_group, 0)
                out_base = (f * VP + wid * V_PW + PH_OFF[ph] * 128) * EMBED
                pltpu.async_copy(
                    out_t.at[pl.ds(0, nv * EMBED)],
                    zrow_hbm.at[pl.ds(out_base, nv * EMBED)],
                    ssem,
                )
            return 0

        lax.fori_loop(0, nf, per_field, 0)
        store_drain(2)
        store_drain(3)

    return _tp_body


def _make_sc_body(nf, f0):
    def _sc_body(x3_hbm, tflat_hbm, *rest):
        outs = rest[:nf]
        idx_v, rows_v, gsem = rest[nf:]
        wid = lax.axis_index("s") * NC + lax.axis_index("c")
        base = wid * BPW
        for i in range(nf):
            pltpu.sync_copy(
                x3_hbm.at[f0 + i, pl.ds(wid * NCHUNK, NCHUNK)], idx_v
            )
            descs = []
            for c in range(NCHUNK):
                descs.append(
                    pltpu.async_copy(
                        tflat_hbm.at[pl.ds(i * VP, VP)].at[idx_v.at[c]],
                        rows_v.at[pl.ds(c * CHUNK, CHUNK)],
                        gsem,
                    )
                )
            for d in descs:
                d.wait()
            pltpu.sync_copy(rows_v, outs[i].at[pl.ds(base, BPW)])

    return _sc_body


@jax.jit
def _embed_all(x3, tablesT):
    mesh = plsc.VectorSubcoreMesh(core_axis_name="c", subcore_axis_name="s")
    outs = []
    f0 = 0
    for nf in GROUP_SIZES:
        ztab = pl.pallas_call(
            _cp_body,
            grid=(nf, NVB // 32),
            in_specs=[
                pl.BlockSpec(
                    (1, EMBED, 4096),
                    functools.partial(lambda f0, f, j: (f0 + f, 0, j), f0),
                )
            ],
            out_specs=pl.BlockSpec(
                (32 * EMBED, 128), lambda f, j: (f * (NVB // 32) + j, 0)
            ),
            out_shape=jax.ShapeDtypeStruct((nf * NVB * EMBED, 128), jnp.float32),
        )(tablesT)

        zrow = pl.kernel(
            _make_tp_body(nf),
            mesh=mesh,
            out_type=jax.ShapeDtypeStruct((nf * VP * EMBED,), jnp.float32),
            scratch_types=[
                pltpu.VMEM((PH_MAX * EMBED, 128), jnp.float32),
                pltpu.VMEM((PH_MAX * EMBED, 128), jnp.float32),
                pltpu.VMEM((PH_MAX * 128 * EMBED,), jnp.float32),
                pltpu.VMEM((PH_MAX * 128 * EMBED,), jnp.float32),
                pltpu.SemaphoreType.DMA,
                pltpu.SemaphoreType.DMA,
            ],
            compiler_params=pltpu.CompilerParams(
                use_tc_tiling_on_sc=False, needs_layout_passes=False
            ),
        )(ztab)
        z2d = zrow.reshape(nf * VP, EMBED)

        g = pl.kernel(
            _make_sc_body(nf, f0),
            mesh=mesh,
            out_type=[jax.ShapeDtypeStruct((BATCH, EMBED), jnp.float32)] * nf,
            scratch_types=[
                pltpu.VMEM((NCHUNK, CHUNK), jnp.int32),
                pltpu.VMEM((BPW, EMBED), jnp.float32),
                pltpu.SemaphoreType.DMA,
            ],
            compiler_params=pltpu.CompilerParams(use_tc_tiling_on_sc=False),
        )
        outs.extend(g(x3, z2d))
        f0 += nf
    return outs


def kernel(x, tables):
    x3 = x.T.reshape(N_FIELDS, BATCH // CHUNK, CHUNK)
    tablesT = jnp.transpose(tables, (0, 2, 1))
    return tuple(_embed_all(x3, tablesT))
